# baseline (speedup 1.0000x reference)
import functools

import jax
import jax.numpy as jnp
from jax.experimental import pallas as pl
from jax.experimental.pallas import tpu as pltpu

_IN = 2048
_OUT = 2048
_TOPK = max(1, int(_OUT * 0.1))
_BT = 256


def _float_keys(r):
    bits = jax.lax.bitcast_convert_type(r, jnp.int32)
    return bits ^ ((bits >> 31) & jnp.int32(0x7FFFFFFF))


def _kth_largest_keys(keys, k):
    rows = keys.shape[0]
    lo = jnp.min(keys, axis=1, keepdims=True)
    hi = jnp.max(keys, axis=1, keepdims=True)
    ones_rhs = jnp.ones((128, 128), jnp.bfloat16)
    kf = jnp.float32(k)

    def body(_, carry):
        lo, hi = carry
        mid = (lo >> 1) + (hi >> 1) + ((lo | hi) & 1)
        total = jnp.zeros((rows, 128), jnp.float32)
        for j in range(16):
            total = total + (keys[:, 128 * j:128 * (j + 1)] >= mid)
        cnt = jax.lax.dot_general(
            total.astype(jnp.bfloat16), ones_rhs, (((1,), (0,)), ((), ())),
            preferred_element_type=jnp.float32)[:, :1]
        eq = cnt == kf
        ge = cnt >= kf
        new_lo = jnp.where(ge, mid, lo)
        new_hi = jnp.where(eq, mid, jnp.where(ge, hi, mid - 1))
        return new_lo, new_hi

    lo, _ = jax.lax.fori_loop(0, 32, body, (lo, hi))
    return lo


def _fused_kernel(x_ref, wr_ref, br_ref, w_ref, b_ref, out_ref):
    xb = x_ref[...]
    dims = (((1,), (1,)), ((), ()))
    r = jax.lax.dot_general(xb, wr_ref[...], dims,
                            preferred_element_type=jnp.float32) + br_ref[...]
    keys = _float_keys(r)
    kth = _kth_largest_keys(keys, _TOPK)
    mask = (keys >= kth).astype(jnp.float32)
    o = jax.lax.dot_general(xb, w_ref[...], dims,
                            preferred_element_type=jnp.float32) + b_ref[...]
    out_ref[...] = o * mask


@jax.jit
def kernel(x, W, b, W_r, b_r):
    B, S, F = x.shape
    T = B * S
    xt = x.reshape(T, F)
    grid = (T // _BT,)
    out = pl.pallas_call(
        _fused_kernel,
        grid=grid,
        in_specs=[
            pl.BlockSpec((_BT, F), lambda i: (i, 0)),
            pl.BlockSpec((_OUT, F), lambda i: (0, 0)),
            pl.BlockSpec((1, _OUT), lambda i: (0, 0)),
            pl.BlockSpec((_OUT, F), lambda i: (0, 0)),
            pl.BlockSpec((1, _OUT), lambda i: (0, 0)),
        ],
        out_specs=pl.BlockSpec((_BT, _OUT), lambda i: (i, 0)),
        out_shape=jax.ShapeDtypeStruct((T, _OUT), jnp.float32),
    )(xt, W_r, b_r.reshape(1, _OUT), W, b.reshape(1, _OUT))
    return out.reshape(B, S, _OUT)


# speedup vs baseline: 1.2247x; 1.2247x over previous
import functools

import jax
import jax.numpy as jnp
from jax.experimental import pallas as pl
from jax.experimental.pallas import tpu as pltpu

_IN = 2048
_OUT = 2048
_TOPK = max(1, int(_OUT * 0.1))
_BT = 256


def _float_keys(r):
    bits = jax.lax.bitcast_convert_type(r, jnp.int32)
    return bits ^ ((bits >> 31) & jnp.int32(0x7FFFFFFF))


def _kth_largest_keys(keys, k):
    rows = keys.shape[0]
    lo = jnp.min(keys, axis=1, keepdims=True)
    hi = jnp.max(keys, axis=1, keepdims=True)

    def cond(carry):
        i, _, _, done = carry
        return jnp.logical_and(i < 33, jnp.logical_not(done))

    def body(carry):
        i, lo, hi, _ = carry
        mid = (lo >> 1) + (hi >> 1) + ((lo | hi) & 1)
        cnt = jnp.sum((keys >= mid).astype(jnp.int32), axis=1, keepdims=True)
        eq = cnt == k
        ge = cnt >= k
        new_lo = jnp.where(ge, mid, lo)
        new_hi = jnp.where(eq, mid, jnp.where(ge, hi, mid - 1))
        done = jnp.all(new_lo >= new_hi)
        return i + 1, new_lo, new_hi, done

    _, lo, _, _ = jax.lax.while_loop(
        cond, body, (jnp.int32(0), lo, hi, jnp.bool_(False)))
    return lo


def _fused_kernel(x_ref, wr_ref, br_ref, w_ref, b_ref, out_ref):
    xb = x_ref[...]
    dims = (((1,), (1,)), ((), ()))
    r = jax.lax.dot_general(xb, wr_ref[...], dims,
                            preferred_element_type=jnp.float32) + br_ref[...]
    keys = _float_keys(r)
    kth = _kth_largest_keys(keys, _TOPK)
    mask = (keys >= kth).astype(jnp.float32)
    o = jax.lax.dot_general(xb, w_ref[...], dims,
                            preferred_element_type=jnp.float32) + b_ref[...]
    out_ref[...] = o * mask


@jax.jit
def kernel(x, W, b, W_r, b_r):
    B, S, F = x.shape
    T = B * S
    xt = x.reshape(T, F)
    grid = (T // _BT,)
    out = pl.pallas_call(
        _fused_kernel,
        grid=grid,
        in_specs=[
            pl.BlockSpec((_BT, F), lambda i: (i, 0)),
            pl.BlockSpec((_OUT, F), lambda i: (0, 0)),
            pl.BlockSpec((1, _OUT), lambda i: (0, 0)),
            pl.BlockSpec((_OUT, F), lambda i: (0, 0)),
            pl.BlockSpec((1, _OUT), lambda i: (0, 0)),
        ],
        out_specs=pl.BlockSpec((_BT, _OUT), lambda i: (i, 0)),
        out_shape=jax.ShapeDtypeStruct((T, _OUT), jnp.float32),
    )(xt, W_r, b_r.reshape(1, _OUT), W, b.reshape(1, _OUT))
    return out.reshape(B, S, _OUT)


# speedup vs baseline: 1.3360x; 1.0909x over previous
import functools

import jax
import jax.numpy as jnp
from jax.experimental import pallas as pl
from jax.experimental.pallas import tpu as pltpu

_IN = 2048
_OUT = 2048
_TOPK = max(1, int(_OUT * 0.1))
_BT = 512


def _float_keys(r):
    bits = jax.lax.bitcast_convert_type(r, jnp.int32)
    return bits ^ ((bits >> 31) & jnp.int32(0x7FFFFFFF))


def _kth_largest_keys(keys, k):
    rows = keys.shape[0]
    lo = jnp.min(keys, axis=1, keepdims=True)
    hi = jnp.max(keys, axis=1, keepdims=True)

    def cond(carry):
        i, _, _, done = carry
        return jnp.logical_and(i < 33, jnp.logical_not(done))

    def body(carry):
        i, lo, hi, _ = carry
        mid = (lo >> 1) + (hi >> 1) + ((lo | hi) & 1)
        cnt = jnp.sum((keys >= mid).astype(jnp.int32), axis=1, keepdims=True)
        eq = cnt == k
        ge = cnt >= k
        new_lo = jnp.where(ge, mid, lo)
        new_hi = jnp.where(eq, mid, jnp.where(ge, hi, mid - 1))
        done = jnp.all(new_lo >= new_hi)
        return i + 1, new_lo, new_hi, done

    _, lo, _, _ = jax.lax.while_loop(
        cond, body, (jnp.int32(0), lo, hi, jnp.bool_(False)))
    return lo


def _fused_kernel(x_ref, wr_ref, br_ref, w_ref, b_ref, out_ref):
    xb = x_ref[...]
    dims = (((1,), (1,)), ((), ()))
    r = jax.lax.dot_general(xb, wr_ref[...], dims,
                            preferred_element_type=jnp.float32) + br_ref[...]
    keys = _float_keys(r)
    kth = _kth_largest_keys(keys, _TOPK)
    mask = (keys >= kth).astype(jnp.float32)
    o = jax.lax.dot_general(xb, w_ref[...], dims,
                            preferred_element_type=jnp.float32) + b_ref[...]
    out_ref[...] = o * mask


@jax.jit
def kernel(x, W, b, W_r, b_r):
    B, S, F = x.shape
    T = B * S
    xt = x.reshape(T, F)
    grid = (T // _BT,)
    out = pl.pallas_call(
        _fused_kernel,
        grid=grid,
        in_specs=[
            pl.BlockSpec((_BT, F), lambda i: (i, 0)),
            pl.BlockSpec((_OUT, F), lambda i: (0, 0)),
            pl.BlockSpec((1, _OUT), lambda i: (0, 0)),
            pl.BlockSpec((_OUT, F), lambda i: (0, 0)),
            pl.BlockSpec((1, _OUT), lambda i: (0, 0)),
        ],
        out_specs=pl.BlockSpec((_BT, _OUT), lambda i: (i, 0)),
        out_shape=jax.ShapeDtypeStruct((T, _OUT), jnp.float32),
    )(xt, W_r, b_r.reshape(1, _OUT), W, b.reshape(1, _OUT))
    return out.reshape(B, S, _OUT)
